# Initial kernel scaffold; baseline (speedup 1.0000x reference)
import jax
import jax.numpy as jnp
from jax.experimental import pallas as pl


def kernel(x, edge_index, W1, b1, W2, b2):
    raise NotImplementedError("write your pallas kernel here")


# speedup vs baseline: 12.9876x; 12.9876x over previous
import functools

import jax
import jax.numpy as jnp
from jax import lax
from jax.experimental import pallas as pl
from jax.experimental.pallas import tpu as pltpu, tpu_sc as plsc

N = 10000
E = 320000
D_IN = 128
N_CLASSES = 64

NC = 2
NS = 16
NW = NC * NS
LANES = 16

B = 128
C = -(-E // (NW * B))
EP = C * B
E_PAD = NW * EP
PAD_DST = N
N_PAD = ((N // (NS * LANES)) + 1) * (NS * LANES)
RPT = N_PAD // NS

_mesh = plsc.VectorSubcoreMesh(core_axis_name="c", subcore_axis_name="s")


@functools.partial(
    pl.kernel,
    out_type=jax.ShapeDtypeStruct((NC, N_PAD), jnp.float32),
    mesh=_mesh,
    scratch_types=[
        pltpu.VMEM((C, B), jnp.int32),
        pltpu.VMEM((B,), jnp.float32),
        pltpu.VMEM_SHARED((N_PAD,), jnp.float32),
    ],
)
def _sc_degree(col_hbm, ones_hbm, z16_hbm, degp_hbm, col_v, ones_v, acc_sh):
    cid = lax.axis_index("c")
    sid = lax.axis_index("s")
    wid = sid * NC + cid
    pltpu.sync_copy(col_hbm.at[wid], col_v)
    pltpu.sync_copy(ones_hbm, ones_v)
    pltpu.sync_copy(z16_hbm.at[pl.ds(sid * RPT, RPT)],
                    acc_sh.at[pl.ds(sid * RPT, RPT)])
    plsc.subcore_barrier()

    @pl.loop(0, C)
    def _count(j):
        pltpu.sync_copy(ones_v, acc_sh.at[col_v.at[j]], add=True)

    plsc.subcore_barrier()
    pltpu.sync_copy(acc_sh.at[pl.ds(sid * RPT, RPT)],
                    degp_hbm.at[cid, pl.ds(sid * RPT, RPT)])


@functools.partial(
    pl.kernel,
    out_type=jax.ShapeDtypeStruct((NC, N_PAD, D_IN), jnp.float32),
    mesh=_mesh,
    scratch_types=[
        pltpu.VMEM((C, B), jnp.int32),
        pltpu.VMEM((C, B), jnp.int32),
        pltpu.VMEM((B, D_IN), jnp.float32),
        pltpu.VMEM_SHARED((N_PAD, D_IN), jnp.float32),
        pltpu.SemaphoreType.DMA,
    ],
)
def _sc_hop(h_hbm, row_hbm, col_hbm, zeros_hbm, out_hbm, row_v, col_v, buf_v, acc_sh, gsem):
    cid = lax.axis_index("c")
    sid = lax.axis_index("s")
    wid = sid * NC + cid

    pltpu.sync_copy(row_hbm.at[wid], row_v)
    pltpu.sync_copy(col_hbm.at[wid], col_v)
    pltpu.sync_copy(zeros_hbm.at[pl.ds(sid * RPT, RPT)],
                    acc_sh.at[pl.ds(sid * RPT, RPT)])
    plsc.subcore_barrier()

    @pl.loop(0, C)
    def _edges(j):
        pltpu.async_copy(h_hbm.at[row_v.at[j]], buf_v, gsem).wait()
        pltpu.sync_copy(buf_v, acc_sh.at[col_v.at[j]], add=True)

    plsc.subcore_barrier()
    pltpu.sync_copy(acc_sh.at[pl.ds(sid * RPT, RPT)],
                    out_hbm.at[cid, pl.ds(sid * RPT, RPT)])


RB = 1280


def _deg_block(degp_blk):
    return degp_blk[0] + degp_blk[1] + 1.0


def _tc_scale_in_body(degp_ref, x_ref, x0_ref):
    deg = _deg_block(degp_ref[...])
    x0_ref[...] = x_ref[...] * lax.rsqrt(deg)[:, None]


def _tc_mid_body(degp_ref, parts_ref, h_ref, y_ref):
    deg = _deg_block(degp_ref[...])
    z = parts_ref[0] + parts_ref[1] + h_ref[...]
    y_ref[...] = z * (1.0 / deg)[:, None]


def _tc_head_body(degp_ref, parts_ref, h_ref, w1_ref, b1_ref, w2_ref, b2_ref, o_ref):
    deg = _deg_block(degp_ref[...])
    z = (parts_ref[0] + parts_ref[1] + h_ref[...]) * lax.rsqrt(deg)[:, None]
    t = jnp.dot(z, w1_ref[...], preferred_element_type=jnp.float32) + b1_ref[...]
    t = jnp.maximum(t, 0.0)
    o_ref[...] = jnp.dot(t, w2_ref[...], preferred_element_type=jnp.float32) + b2_ref[...]


_degp_spec = pl.BlockSpec((NC, RB), lambda i: (0, i))
_row_spec = pl.BlockSpec((RB, D_IN), lambda i: (i, 0))
_parts_spec = pl.BlockSpec((NC, RB, D_IN), lambda i: (0, i, 0))

_tc_scale_in = pl.pallas_call(
    _tc_scale_in_body,
    grid=(N_PAD // RB,),
    in_specs=[_degp_spec, _row_spec],
    out_specs=_row_spec,
    out_shape=jax.ShapeDtypeStruct((N_PAD, D_IN), jnp.float32),
)

_tc_mid = pl.pallas_call(
    _tc_mid_body,
    grid=(N_PAD // RB,),
    in_specs=[_degp_spec, _parts_spec, _row_spec],
    out_specs=_row_spec,
    out_shape=jax.ShapeDtypeStruct((N_PAD, D_IN), jnp.float32),
)

_tc_head = pl.pallas_call(
    _tc_head_body,
    grid=(N_PAD // RB,),
    in_specs=[
        _degp_spec,
        _parts_spec,
        _row_spec,
        pl.BlockSpec((D_IN, D_IN), lambda i: (0, 0)),
        pl.BlockSpec((1, D_IN), lambda i: (0, 0)),
        pl.BlockSpec((D_IN, N_CLASSES), lambda i: (0, 0)),
        pl.BlockSpec((1, N_CLASSES), lambda i: (0, 0)),
    ],
    out_specs=pl.BlockSpec((RB, N_CLASSES), lambda i: (i, 0)),
    out_shape=jax.ShapeDtypeStruct((N_PAD, N_CLASSES), jnp.float32),
)


def kernel(x, edge_index, W1, b1, W2, b2):
    row = edge_index[0].astype(jnp.int32)
    col = edge_index[1].astype(jnp.int32)
    pad = E_PAD - E
    rowp = jnp.concatenate([row, jnp.zeros((pad,), jnp.int32)]).reshape(NW, C, B)
    colp = jnp.concatenate([col, jnp.full((pad,), PAD_DST, jnp.int32)]).reshape(NW, C, B)
    zeros = jnp.zeros((N_PAD, D_IN), jnp.float32)

    ones1 = jnp.ones((B,), jnp.float32)
    zeros1 = jnp.zeros((N_PAD,), jnp.float32)
    degp = _sc_degree(colp, ones1, zeros1)
    xp = jnp.pad(x, ((0, N_PAD - N), (0, 0)))
    x0 = _tc_scale_in(degp, xp)
    parts1 = _sc_hop(x0, rowp, colp, zeros)
    y1 = _tc_mid(degp, parts1, x0)
    parts2 = _sc_hop(y1, rowp, colp, zeros)
    out = _tc_head(degp, parts2, y1,
                   W1, b1.reshape(1, D_IN), W2, b2.reshape(1, N_CLASSES))
    return out[:N]
